# Initial kernel scaffold; baseline (speedup 1.0000x reference)
import jax
import jax.numpy as jnp
from jax import lax
from jax.experimental import pallas as pl
from jax.experimental.pallas import tpu as pltpu
from jax.experimental.pallas import tpu_sc as plsc

N = 131072
NU = 1024
D = 64
C = 3
NS = 2
NE = 1
K = C * (NS + 1) + NE

L = 16
NW = 32
NG = N // L
GPW = NG // NW
BG = 8
NBLK = GPW // BG
NEG = jnp.float32(-1e30)

_mesh = plsc.VectorSubcoreMesh(core_axis_name="c", subcore_axis_name="s")


def _body(feat_hbm, means_hbm, lp_hbm, labels_hbm, cn_hbm, usn_hbm, ex_hbm,
          tv_out, nc_out, counts_out,
          means_v, cn_v, usn_v, bias_v, feat_v, lab_v, ex_v,
          tv_stage, nc_stage, banks_v, merged_v):
  wid = lax.axis_index("s") * 2 + lax.axis_index("c")
  iota = lax.iota(jnp.int32, L)
  ones = jnp.ones((L,), jnp.float32)
  zf = jnp.zeros((L,), jnp.float32)
  zi = jnp.zeros((L,), jnp.int32)

  pltpu.sync_copy(means_hbm, means_v)
  pltpu.sync_copy(cn_hbm, cn_v)
  pltpu.sync_copy(usn_hbm, usn_v)
  pltpu.sync_copy(lp_hbm, bias_v)

  @pl.loop(0, NU // L)
  def _bias(uc):
    base = uc * L * D + iota * D
    def bstep(d, acc):
      m = plsc.load_gather(means_v, [base + d])
      return acc + m * m
    sq = lax.fori_loop(0, D, bstep, zf)
    lp = bias_v[pl.ds(uc * L, L)]
    bias_v[pl.ds(uc * L, L)] = lp - 0.5 * sq

  @pl.loop(0, NU)
  def _zero(i):
    banks_v[pl.ds(i * L, L)] = zf

  @pl.loop(0, NBLK)
  def _block(b):
    gbase = (wid * NBLK + b) * BG
    sbase = gbase * L
    pltpu.sync_copy(feat_hbm.at[pl.ds(gbase, BG)], feat_v)
    pltpu.sync_copy(labels_hbm.at[pl.ds(sbase, BG * L)], lab_v)
    pltpu.sync_copy(ex_hbm.at[pl.ds(sbase, BG * L)], ex_v)

    @pl.loop(0, BG)
    def _group(bg):
      labs = lab_v[pl.ds(bg * L, L)]
      cands = []
      tops = []
      for c in range(C):
        t = plsc.load_gather(cn_v, [labs * C + c])
        tops.append(t)
        cands.append(t)
      for c in range(C):
        for s in range(NS):
          cands.append(plsc.load_gather(usn_v, [tops[c] * NS + s]))
      cands.append(ex_v[pl.ds(bg * L, L)])

      bias = [plsc.load_gather(bias_v, [cands[k]]) for k in range(K)]
      cbase = [cands[k] * D for k in range(K)]

      def dstep(d, carry):
        xacc = carry[0]
        accs = list(carry[1:])
        f = feat_v[bg, d, :]
        xacc = xacc + f * f
        for k in range(K):
          m = plsc.load_gather(means_v, [cbase[k] + d])
          accs[k] = accs[k] + f * m
        return (xacc,) + tuple(accs)

      res = lax.fori_loop(0, D, dstep, (zf,) * (K + 1))
      xn = res[0]
      scores = [res[1 + k] - 0.5 * xn + bias[k] for k in range(K)]

      for j in range(C):
        best_v = scores[0]
        best_k = zi
        for k in range(1, K):
          take = scores[k] > best_v
          best_v = jnp.where(take, scores[k], best_v)
          best_k = jnp.where(take, jnp.int32(k), best_k)
        best_u = zi
        for k in range(K):
          is_k = best_k == k
          best_u = jnp.where(is_k, cands[k], best_u)
          if j < C - 1:
            scores[k] = jnp.where(is_k, NEG, scores[k])
        tv_stage[bg, j, :] = best_v
        nc_stage[bg, j, :] = best_u
        plsc.addupdate_scatter(banks_v, [iota * NU + best_u], ones)

    pltpu.sync_copy(tv_stage, tv_out.at[pl.ds(gbase, BG)])
    pltpu.sync_copy(nc_stage, nc_out.at[pl.ds(gbase, BG)])

  @pl.loop(0, NU // L)
  def _merge(uc):
    acc = zf
    for b in range(L):
      acc = acc + banks_v[pl.ds(b * NU + uc * L, L)]
    merged_v[pl.ds(uc * L, L)] = acc

  pltpu.sync_copy(merged_v, counts_out.at[wid])


_sc_call = pl.kernel(
    _body,
    out_type=(
        jax.ShapeDtypeStruct((NG, C, L), jnp.float32),
        jax.ShapeDtypeStruct((NG, C, L), jnp.int32),
        jax.ShapeDtypeStruct((NW, NU), jnp.float32),
    ),
    mesh=_mesh,
    scratch_types=[
        pltpu.VMEM((NU * D,), jnp.float32),
        pltpu.VMEM((NU * C,), jnp.int32),
        pltpu.VMEM((NU * NS,), jnp.int32),
        pltpu.VMEM((NU,), jnp.float32),
        pltpu.VMEM((BG, D, L), jnp.float32),
        pltpu.VMEM((BG * L,), jnp.int32),
        pltpu.VMEM((BG * L,), jnp.int32),
        pltpu.VMEM((BG, C, L), jnp.float32),
        pltpu.VMEM((BG, C, L), jnp.int32),
        pltpu.VMEM((L * NU,), jnp.float32),
        pltpu.VMEM((NU,), jnp.float32),
    ],
)


@jax.jit
def kernel(features, means, log_proportions, labels, closest_neighbors,
           unit_search_neighbors, explore_ids):
  feat_g = features.reshape(NG, L, D).transpose(0, 2, 1)
  tv, nc, cparts = _sc_call(
      feat_g,
      means.reshape(-1),
      log_proportions,
      labels.astype(jnp.int32),
      closest_neighbors.reshape(-1).astype(jnp.int32),
      unit_search_neighbors.reshape(-1).astype(jnp.int32),
      explore_ids.reshape(-1).astype(jnp.int32),
  )
  top_vals = tv.transpose(0, 2, 1).reshape(N, C)
  new_cand = nc.transpose(0, 2, 1).reshape(N, C)
  counts = cparts.sum(axis=0)
  return top_vals, new_cand, counts


# speedup vs baseline: 12.4865x; 12.4865x over previous
import jax
import jax.numpy as jnp
from jax import lax
from jax.experimental import pallas as pl
from jax.experimental.pallas import tpu as pltpu
from jax.experimental.pallas import tpu_sc as plsc

N = 131072
NU = 1024
D = 64
C = 3
NS = 2
NE = 1
K = C * (NS + 1) + NE

L = 16
NW = 32
NG = N // L
GPW = NG // NW
BG = 8
NBLK = GPW // BG
NEG = -1e30

_mesh = plsc.VectorSubcoreMesh(core_axis_name="c", subcore_axis_name="s")


def _body(feat_hbm, means_hbm, lp_hbm, labels_hbm, cn_hbm, usn_hbm, ex_hbm,
          tv_out, nc_out, counts_out,
          means_v, cn_v, usn_v, bias_v, feat_v, lab_v, ex_v,
          tv_stage, nc_stage, banks_v, merged_v):
  wid = lax.axis_index("s") * 2 + lax.axis_index("c")
  iota = lax.iota(jnp.int32, L)
  ones = jnp.ones((L,), jnp.float32)
  zf = jnp.zeros((L,), jnp.float32)
  zi = jnp.zeros((L,), jnp.int32)

  pltpu.sync_copy(means_hbm, means_v)
  pltpu.sync_copy(cn_hbm, cn_v)
  pltpu.sync_copy(usn_hbm, usn_v)
  pltpu.sync_copy(lp_hbm, bias_v)

  @pl.loop(0, NU // L)
  def _bias(uc):
    base = uc * L * D + iota * D
    def bstep(d, acc):
      m = plsc.load_gather(means_v, [base + d])
      return acc + m * m
    sq = lax.fori_loop(0, D, bstep, zf)
    lp = bias_v[pl.ds(uc * L, L)]
    bias_v[pl.ds(uc * L, L)] = lp - 0.5 * sq

  @pl.loop(0, NU)
  def _zero(i):
    banks_v[pl.ds(i * L, L)] = zf

  @pl.loop(0, NBLK)
  def _block(b):
    gbase = (wid * NBLK + b) * BG
    sbase = gbase * L
    pltpu.sync_copy(feat_hbm.at[pl.ds(gbase * D * L, BG * D * L)], feat_v)
    pltpu.sync_copy(labels_hbm.at[pl.ds(sbase, BG * L)], lab_v)
    pltpu.sync_copy(ex_hbm.at[pl.ds(sbase, BG * L)], ex_v)

    @pl.loop(0, BG)
    def _group(bg):
      labs = lab_v[pl.ds(bg * L, L)]
      cands = []
      tops = []
      for c in range(C):
        t = plsc.load_gather(cn_v, [labs * C + c])
        tops.append(t)
        cands.append(t)
      for c in range(C):
        for s in range(NS):
          cands.append(plsc.load_gather(usn_v, [tops[c] * NS + s]))
      cands.append(ex_v[pl.ds(bg * L, L)])

      bias = [plsc.load_gather(bias_v, [cands[k]]) for k in range(K)]
      cbase = [cands[k] * D for k in range(K)]

      def dstep(d, carry):
        xacc = carry[0]
        accs = list(carry[1:])
        f = feat_v[pl.ds((bg * D + d) * L, L)]
        xacc = xacc + f * f
        for k in range(K):
          m = plsc.load_gather(means_v, [cbase[k] + d])
          accs[k] = accs[k] + f * m
        return (xacc,) + tuple(accs)

      res = lax.fori_loop(0, D, dstep, (zf,) * (K + 1))
      xn = res[0]
      scores = [res[1 + k] - 0.5 * xn + bias[k] for k in range(K)]

      for j in range(C):
        best_v = scores[0]
        best_k = zi
        for k in range(1, K):
          take = scores[k] > best_v
          best_v = jnp.where(take, scores[k], best_v)
          best_k = jnp.where(take, jnp.int32(k), best_k)
        best_u = zi
        for k in range(K):
          is_k = best_k == k
          best_u = jnp.where(is_k, cands[k], best_u)
          if j < C - 1:
            scores[k] = jnp.where(is_k, NEG, scores[k])
        tv_stage[pl.ds((bg * C + j) * L, L)] = best_v
        nc_stage[pl.ds((bg * C + j) * L, L)] = best_u
        plsc.addupdate_scatter(banks_v, [iota * NU + best_u], ones)

    pltpu.sync_copy(tv_stage, tv_out.at[pl.ds(gbase * C * L, BG * C * L)])
    pltpu.sync_copy(nc_stage, nc_out.at[pl.ds(gbase * C * L, BG * C * L)])

  @pl.loop(0, NU // L)
  def _merge(uc):
    acc = zf
    for b in range(L):
      acc = acc + banks_v[pl.ds(b * NU + uc * L, L)]
    merged_v[pl.ds(uc * L, L)] = acc

  pltpu.sync_copy(merged_v, counts_out.at[wid])


_sc_call = pl.kernel(
    _body,
    out_type=(
        jax.ShapeDtypeStruct((NG * C * L,), jnp.float32),
        jax.ShapeDtypeStruct((NG * C * L,), jnp.int32),
        jax.ShapeDtypeStruct((NW, NU), jnp.float32),
    ),
    mesh=_mesh,
    compiler_params=pltpu.CompilerParams(needs_layout_passes=False),
    scratch_types=[
        pltpu.VMEM((NU * D,), jnp.float32),
        pltpu.VMEM((NU * C,), jnp.int32),
        pltpu.VMEM((NU * NS,), jnp.int32),
        pltpu.VMEM((NU,), jnp.float32),
        pltpu.VMEM((BG * D * L,), jnp.float32),
        pltpu.VMEM((BG * L,), jnp.int32),
        pltpu.VMEM((BG * L,), jnp.int32),
        pltpu.VMEM((BG * C * L,), jnp.float32),
        pltpu.VMEM((BG * C * L,), jnp.int32),
        pltpu.VMEM((L * NU,), jnp.float32),
        pltpu.VMEM((NU,), jnp.float32),
    ],
)


@jax.jit
def kernel(features, means, log_proportions, labels, closest_neighbors,
           unit_search_neighbors, explore_ids):
  feat_g = features.reshape(NG, L, D).transpose(0, 2, 1).reshape(-1)
  tv, nc, cparts = _sc_call(
      feat_g,
      means.reshape(-1),
      log_proportions,
      labels.astype(jnp.int32),
      closest_neighbors.reshape(-1).astype(jnp.int32),
      unit_search_neighbors.reshape(-1).astype(jnp.int32),
      explore_ids.reshape(-1).astype(jnp.int32),
  )
  top_vals = tv.reshape(NG, C, L).transpose(0, 2, 1).reshape(N, C)
  new_cand = nc.reshape(NG, C, L).transpose(0, 2, 1).reshape(N, C)
  counts = cparts.sum(axis=0)
  return top_vals, new_cand, counts
